# Initial kernel scaffold; baseline (speedup 1.0000x reference)
import jax
import jax.numpy as jnp
from jax.experimental import pallas as pl


def kernel(x, edge_index, W1, b1, W2, b2):
    raise NotImplementedError("write your pallas kernel here")


# speedup vs baseline: 9.6790x; 9.6790x over previous
import functools

import jax
import jax.numpy as jnp
from jax import lax
from jax.experimental import pallas as pl
from jax.experimental.pallas import tpu as pltpu
from jax.experimental.pallas import tpu_sc as plsc

_CHUNK = 80
_F32 = jnp.float32
_TILES = 16
_BM = 640


def _make_deg(n, e):
    tiles = _TILES
    rpt = n // tiles
    ept = e // (2 * tiles)
    chunks = ept // _CHUNK
    mesh = plsc.VectorSubcoreMesh(core_axis_name="c", subcore_axis_name="s")

    @functools.partial(
        pl.kernel,
        mesh=mesh,
        out_type=jax.ShapeDtypeStruct((2 * n, 128), _F32),
        scratch_types=[
            pltpu.VMEM((_CHUNK,), jnp.int32),
            pltpu.VMEM((_CHUNK, 128), _F32),
            pltpu.VMEM_SHARED((n, 128), _F32),
        ],
    )
    def deg(dst, ones, deg_cat, didx, ones_v, acc):
        cid = lax.axis_index("c")
        sid = lax.axis_index("s")
        r0 = sid * rpt
        pltpu.sync_copy(ones, ones_v)
        for r in range(rpt // _CHUNK):
            pltpu.sync_copy(ones_v, acc.at[pl.ds(r0 + r * _CHUNK, _CHUNK), :])
        plsc.subcore_barrier()
        base = (cid * tiles + sid) * ept

        def body(k, carry):
            off = pl.multiple_of(base + k * _CHUNK, 8)
            pltpu.sync_copy(dst.at[pl.ds(off, _CHUNK)], didx)
            pltpu.sync_copy(ones_v, acc.at[didx], add=True)
            return carry

        lax.fori_loop(0, chunks, body, None)
        plsc.subcore_barrier()
        ob = pl.multiple_of(cid * n + r0, 8)
        pltpu.sync_copy(acc.at[pl.ds(r0, rpt), :], deg_cat.at[pl.ds(ob, rpt), :])

    return deg


def _make_agg_cols(n, e, dh):
    tiles = _TILES
    rpt = n // tiles
    ept = e // tiles
    chunks = ept // _CHUNK
    mesh = plsc.VectorSubcoreMesh(core_axis_name="c", subcore_axis_name="s")

    @functools.partial(
        pl.kernel,
        mesh=mesh,
        out_type=jax.ShapeDtypeStruct((2 * n, dh), _F32),
        scratch_types=[
            pltpu.VMEM((_CHUNK,), jnp.int32),
            pltpu.VMEM((_CHUNK,), jnp.int32),
            pltpu.VMEM((_CHUNK, dh), _F32),
            pltpu.VMEM_SHARED((n, dh), _F32),
            pltpu.SemaphoreType.DMA,
        ],
    )
    def agg(hs_cat, src_cat, dst, out_cat, sidx, didx, rows, acc, sem):
        cid = lax.axis_index("c")
        sid = lax.axis_index("s")
        r0 = sid * rpt
        tb = pl.multiple_of(cid * n + r0, 8)
        pltpu.sync_copy(hs_cat.at[pl.ds(tb, rpt), :], acc.at[pl.ds(r0, rpt), :])
        plsc.subcore_barrier()
        ibase = cid * e + sid * ept
        dbase = sid * ept

        def body(k, carry):
            off = pl.multiple_of(ibase + k * _CHUNK, 8)
            doff = pl.multiple_of(dbase + k * _CHUNK, 8)
            pltpu.sync_copy(src_cat.at[pl.ds(off, _CHUNK)], sidx)
            pltpu.sync_copy(dst.at[pl.ds(doff, _CHUNK)], didx)
            pltpu.async_copy(hs_cat.at[sidx], rows, sem).wait()
            pltpu.sync_copy(rows, acc.at[didx], add=True)
            return carry

        lax.fori_loop(0, chunks, body, None)
        plsc.subcore_barrier()
        pltpu.sync_copy(acc.at[pl.ds(r0, rpt), :], out_cat.at[pl.ds(tb, rpt), :])

    return agg


def _make_agg_edges(n, e, dh):
    tiles = _TILES
    rpt = n // tiles
    ept = e // (2 * tiles)
    chunks = ept // _CHUNK
    mesh = plsc.VectorSubcoreMesh(core_axis_name="c", subcore_axis_name="s")

    @functools.partial(
        pl.kernel,
        mesh=mesh,
        out_type=jax.ShapeDtypeStruct((2 * n, dh), _F32),
        scratch_types=[
            pltpu.VMEM((_CHUNK,), jnp.int32),
            pltpu.VMEM((_CHUNK,), jnp.int32),
            pltpu.VMEM((_CHUNK, dh), _F32),
            pltpu.VMEM_SHARED((n, dh), _F32),
            pltpu.SemaphoreType.DMA,
        ],
    )
    def agg(hs, src, dst, out_cat, sidx, didx, rows, acc, sem):
        cid = lax.axis_index("c")
        sid = lax.axis_index("s")
        r0 = sid * rpt
        pltpu.sync_copy(hs.at[pl.ds(r0, rpt), :], acc.at[pl.ds(r0, rpt), :])
        plsc.subcore_barrier()
        base = (cid * tiles + sid) * ept

        def body(k, carry):
            off = pl.multiple_of(base + k * _CHUNK, 8)
            pltpu.sync_copy(src.at[pl.ds(off, _CHUNK)], sidx)
            pltpu.sync_copy(dst.at[pl.ds(off, _CHUNK)], didx)
            pltpu.async_copy(hs.at[sidx], rows, sem).wait()
            pltpu.sync_copy(rows, acc.at[didx], add=True)
            return carry

        lax.fori_loop(0, chunks, body, None)
        plsc.subcore_barrier()
        ob = pl.multiple_of(cid * n + r0, 8)
        pltpu.sync_copy(acc.at[pl.ds(r0, rpt), :], out_cat.at[pl.ds(ob, rpt), :])

    return agg


def _dinv_of(d0, d1):
    return lax.rsqrt(d0[:, 0:1] + d1[:, 0:1] - 1.0)


def _tc1(x, w1, deg0, deg1):
    n, din = x.shape
    dh2 = w1.shape[1]
    half = dh2 // 2
    npb = n // _BM

    def body(x_ref, w_ref, d0_ref, d1_ref, o_ref):
        dinv = _dinv_of(d0_ref[...], d1_ref[...])
        o_ref[...] = jnp.dot(x_ref[...], w_ref[...], preferred_element_type=_F32) * dinv

    return pl.pallas_call(
        body,
        grid=(2, npb),
        in_specs=[
            pl.BlockSpec((_BM, din), lambda c, i: (i, 0)),
            pl.BlockSpec((din, half), lambda c, i: (0, c)),
            pl.BlockSpec((_BM, 128), lambda c, i: (i, 0)),
            pl.BlockSpec((_BM, 128), lambda c, i: (i, 0)),
        ],
        out_specs=pl.BlockSpec((_BM, half), lambda c, i: (c * npb + i, 0)),
        out_shape=jax.ShapeDtypeStruct((2 * n, half), _F32),
    )(x, w1, deg0, deg1)


def _tc2(acc0, acc1, deg0, deg1, b1, w2):
    n, half1 = acc0.shape
    dh = 2 * half1
    dout = w2.shape[1]

    def body(a0_ref, a1_ref, d0_ref, d1_ref, b_ref, w_ref, o_ref):
        dinv = _dinv_of(d0_ref[...], d1_ref[...])
        a = jnp.concatenate([a0_ref[...], a1_ref[...]], axis=1)
        h1 = jnp.maximum(a * dinv + b_ref[...], 0.0)
        o_ref[...] = jnp.dot(h1, w_ref[...], preferred_element_type=_F32) * dinv

    return pl.pallas_call(
        body,
        grid=(n // _BM,),
        in_specs=[
            pl.BlockSpec((_BM, half1), lambda i: (i, 0)),
            pl.BlockSpec((_BM, half1), lambda i: (i, 0)),
            pl.BlockSpec((_BM, 128), lambda i: (i, 0)),
            pl.BlockSpec((_BM, 128), lambda i: (i, 0)),
            pl.BlockSpec((1, dh), lambda i: (0, 0)),
            pl.BlockSpec((dh, dout), lambda i: (0, 0)),
        ],
        out_specs=pl.BlockSpec((_BM, dout), lambda i: (i, 0)),
        out_shape=jax.ShapeDtypeStruct((n, dout), _F32),
    )(acc0, acc1, deg0, deg1, b1, w2)


def _tc3(p0, p1, hs2, deg0, deg1, b2):
    n, dout = p0.shape

    def body(a0_ref, a1_ref, h_ref, d0_ref, d1_ref, b_ref, o_ref):
        dinv = _dinv_of(d0_ref[...], d1_ref[...])
        a = a0_ref[...] + a1_ref[...] - h_ref[...]
        o_ref[...] = a * dinv + b_ref[...]

    return pl.pallas_call(
        body,
        grid=(n // _BM,),
        in_specs=[
            pl.BlockSpec((_BM, dout), lambda i: (i, 0)),
            pl.BlockSpec((_BM, dout), lambda i: (i, 0)),
            pl.BlockSpec((_BM, dout), lambda i: (i, 0)),
            pl.BlockSpec((_BM, 128), lambda i: (i, 0)),
            pl.BlockSpec((_BM, 128), lambda i: (i, 0)),
            pl.BlockSpec((1, dout), lambda i: (0, 0)),
        ],
        out_specs=pl.BlockSpec((_BM, dout), lambda i: (i, 0)),
        out_shape=jax.ShapeDtypeStruct((n, dout), _F32),
    )(p0, p1, hs2, deg0, deg1, b2)


def kernel(x, edge_index, W1, b1, W2, b2):
    n = x.shape[0]
    e = edge_index.shape[1]
    src = edge_index[0]
    dst = edge_index[1]

    step = _TILES * _BM
    np_ = ((n + step - 1) // step) * step
    x_p = jnp.pad(x, ((0, np_ - n), (0, 0)))

    ones = jnp.ones((_CHUNK, 128), _F32)
    deg_cat = _make_deg(np_, e)(dst, ones)
    deg0, deg1 = deg_cat[:np_], deg_cat[np_:]

    hs_cat = _tc1(x_p, W1, deg0, deg1)
    src_cat = jnp.concatenate([src, src + np_])
    acc_cat = _make_agg_cols(np_, e, W1.shape[1] // 2)(hs_cat, src_cat, dst)
    acc0, acc1 = acc_cat[:np_], acc_cat[np_:]

    hs2 = _tc2(acc0, acc1, deg0, deg1, b1.reshape(1, -1), W2)
    p_cat = _make_agg_edges(np_, e, W2.shape[1])(hs2, src, dst)
    p0, p1 = p_cat[:np_], p_cat[np_:]
    return _tc3(p0, p1, hs2, deg0, deg1, b2.reshape(1, -1))[:n]


# speedup vs baseline: 10.7582x; 1.1115x over previous
import functools

import jax
import jax.numpy as jnp
from jax import lax
from jax.experimental import pallas as pl
from jax.experimental.pallas import tpu as pltpu
from jax.experimental.pallas import tpu_sc as plsc

_CHUNK = 128
_F32 = jnp.float32
_TILES = 16
_BM = 640
_NBUF = 2


def _agg_body(hs, acc, srcp1, dstp, sidx, didx_all, rbufs, sems,
              sbase, dbase, chunks):
    g = len(rbufs)
    _H = 80

    def do_pair(h, j, nloc):
        for b in range(g):
            cl = g * j + b
            cg = h * _H + cl
            pltpu.sync_copy(srcp1.at[pl.ds((sbase + cg) * _CHUNK, _CHUNK)], sidx[b])
            pltpu.async_copy(hs.at[sidx[b]], rbufs[b], sems[b])
        for b in range(g):
            cl = g * j + b
            pltpu.make_async_copy(hs.at[sidx[b]], rbufs[b], sems[b]).wait()
            pltpu.sync_copy(rbufs[b], acc.at[didx_all.at[cl]], add=True)

    def do_one(h, cl):
        cg = h * _H + cl
        pltpu.sync_copy(srcp1.at[pl.ds((sbase + cg) * _CHUNK, _CHUNK)], sidx[0])
        pltpu.async_copy(hs.at[sidx[0]], rbufs[0], sems[0])
        pltpu.make_async_copy(hs.at[sidx[0]], rbufs[0], sems[0]).wait()
        pltpu.sync_copy(rbufs[0], acc.at[didx_all.at[cl]], add=True)

    for h in range(-(-chunks // _H)):
        nloc = min(_H, chunks - h * _H)
        pltpu.sync_copy(dstp.at[pl.ds(pl.multiple_of(dbase + h * _H, 8), _H), :],
                        didx_all)
        m = nloc // g

        def body(j, carry, h=h, nloc=nloc):
            do_pair(h, j, nloc)
            return carry

        lax.fori_loop(0, m, body, None)
        for cl in range(g * m, nloc):
            do_one(h, cl)


def _agg_scratch(n, dh):
    return [
        pltpu.VMEM((_CHUNK,), jnp.int32),
        pltpu.VMEM((_CHUNK,), jnp.int32),
        pltpu.VMEM((80, _CHUNK), jnp.int32),
    ] + [pltpu.VMEM((_CHUNK, dh), _F32)] * _NBUF + [
        pltpu.VMEM_SHARED((n + 8, dh), _F32),
    ] + [pltpu.SemaphoreType.DMA] * _NBUF


def _make_agg_cols(n, e, dh):
    tiles = _TILES
    rpt = n // tiles
    ept = e // tiles
    chunks = -(-ept // _CHUNK)
    cpad = -(-chunks // 8) * 8
    mesh = plsc.VectorSubcoreMesh(core_axis_name="c", subcore_axis_name="s")

    @functools.partial(
        pl.kernel,
        mesh=mesh,
        out_type=jax.ShapeDtypeStruct((2 * n, dh), _F32),
        scratch_types=_agg_scratch(n, dh),
    )
    def agg(hs_cat, srcp1, dstp, out_cat, si0, si1, didx_all, *rest):
        rbufs, (acc,), sems = rest[:_NBUF], rest[_NBUF:_NBUF + 1], rest[_NBUF + 1:]
        cid = lax.axis_index("c")
        sid = lax.axis_index("s")
        r0 = sid * rpt
        tb = pl.multiple_of(cid * n + r0, 8)
        pltpu.sync_copy(hs_cat.at[pl.ds(tb, rpt), :], acc.at[pl.ds(r0, rpt), :])
        plsc.subcore_barrier()
        sbase = pl.multiple_of(cid * tiles * cpad + sid * cpad, 8)
        dbase = pl.multiple_of(sid * cpad, 8)
        _agg_body(hs_cat, acc, srcp1, dstp, (si0, si1), didx_all, rbufs, sems,
                  sbase, dbase, chunks)
        plsc.subcore_barrier()
        pltpu.sync_copy(acc.at[pl.ds(r0, rpt), :], out_cat.at[pl.ds(tb, rpt), :])

    return agg


def _make_agg_edges(n, e, dh):
    tiles = _TILES
    rpt = n // tiles
    ept = e // (2 * tiles)
    chunks = -(-ept // _CHUNK)
    cpad = -(-chunks // 8) * 8
    mesh = plsc.VectorSubcoreMesh(core_axis_name="c", subcore_axis_name="s")

    @functools.partial(
        pl.kernel,
        mesh=mesh,
        out_type=jax.ShapeDtypeStruct((2 * n, dh), _F32),
        scratch_types=_agg_scratch(n, dh),
    )
    def agg(hs, srcp1, dstp, out_cat, si0, si1, didx_all, *rest):
        rbufs, (acc,), sems = rest[:_NBUF], rest[_NBUF:_NBUF + 1], rest[_NBUF + 1:]
        cid = lax.axis_index("c")
        sid = lax.axis_index("s")
        r0 = sid * rpt
        pltpu.sync_copy(hs.at[pl.ds(r0, rpt), :], acc.at[pl.ds(r0, rpt), :])
        plsc.subcore_barrier()
        base = pl.multiple_of((cid * tiles + sid) * cpad, 8)
        _agg_body(hs, acc, srcp1, dstp, (si0, si1), didx_all, rbufs, sems,
                  base, base, chunks)
        plsc.subcore_barrier()
        ob = pl.multiple_of(cid * n + r0, 8)
        pltpu.sync_copy(acc.at[pl.ds(r0, rpt), :], out_cat.at[pl.ds(ob, rpt), :])

    return agg


def _dinv_of(d0, d1):
    return lax.rsqrt(d0[:, 0:1] + d1[:, 0:1] - 1.0)


def _tc1(x, w1, deg0, deg1):
    n, din = x.shape
    dh2 = w1.shape[1]
    half = dh2 // 2
    npb = n // _BM

    def body(x_ref, w_ref, d0_ref, d1_ref, o_ref):
        dinv = _dinv_of(d0_ref[...], d1_ref[...])
        o_ref[...] = jnp.dot(x_ref[...], w_ref[...], preferred_element_type=_F32) * dinv

    return pl.pallas_call(
        body,
        grid=(2, npb),
        in_specs=[
            pl.BlockSpec((_BM, din), lambda c, i: (i, 0)),
            pl.BlockSpec((din, half), lambda c, i: (0, c)),
            pl.BlockSpec((_BM, 128), lambda c, i: (i, 0)),
            pl.BlockSpec((_BM, 128), lambda c, i: (i, 0)),
        ],
        out_specs=pl.BlockSpec((_BM, half), lambda c, i: (c * npb + i, 0)),
        out_shape=jax.ShapeDtypeStruct((2 * n, half), _F32),
    )(x, w1, deg0, deg1)


def _tc2(acc0, acc1, deg0, deg1, b1, w2):
    n, half1 = acc0.shape
    dh = 2 * half1
    dout = w2.shape[1]

    def body(a0_ref, a1_ref, d0_ref, d1_ref, b_ref, w_ref, o_ref):
        dinv = _dinv_of(d0_ref[...], d1_ref[...])
        a = jnp.concatenate([a0_ref[...], a1_ref[...]], axis=1)
        h1 = jnp.maximum(a * dinv + b_ref[...], 0.0)
        o_ref[...] = jnp.dot(h1, w_ref[...], preferred_element_type=_F32) * dinv

    return pl.pallas_call(
        body,
        grid=(n // _BM,),
        in_specs=[
            pl.BlockSpec((_BM, half1), lambda i: (i, 0)),
            pl.BlockSpec((_BM, half1), lambda i: (i, 0)),
            pl.BlockSpec((_BM, 128), lambda i: (i, 0)),
            pl.BlockSpec((_BM, 128), lambda i: (i, 0)),
            pl.BlockSpec((1, dh), lambda i: (0, 0)),
            pl.BlockSpec((dh, dout), lambda i: (0, 0)),
        ],
        out_specs=pl.BlockSpec((_BM, dout), lambda i: (i, 0)),
        out_shape=jax.ShapeDtypeStruct((n, dout), _F32),
    )(acc0, acc1, deg0, deg1, b1, w2)


def _tc3(p0, p1, hs2, deg0, deg1, b2):
    n, dout = p0.shape

    def body(a0_ref, a1_ref, h_ref, d0_ref, d1_ref, b_ref, o_ref):
        dinv = _dinv_of(d0_ref[...], d1_ref[...])
        a = a0_ref[...] + a1_ref[...] - h_ref[...]
        o_ref[...] = a * dinv + b_ref[...]

    return pl.pallas_call(
        body,
        grid=(n // _BM,),
        in_specs=[
            pl.BlockSpec((_BM, dout), lambda i: (i, 0)),
            pl.BlockSpec((_BM, dout), lambda i: (i, 0)),
            pl.BlockSpec((_BM, dout), lambda i: (i, 0)),
            pl.BlockSpec((_BM, 128), lambda i: (i, 0)),
            pl.BlockSpec((_BM, 128), lambda i: (i, 0)),
            pl.BlockSpec((1, dout), lambda i: (0, 0)),
        ],
        out_specs=pl.BlockSpec((_BM, dout), lambda i: (i, 0)),
        out_shape=jax.ShapeDtypeStruct((n, dout), _F32),
    )(p0, p1, hs2, deg0, deg1, b2)


def _tile_pad_idx(a, tiles, ch, fill):
    ept = a.shape[0] // tiles
    chunks = -(-ept // ch)
    cpad = -(-chunks // 8) * 8
    a2 = a.reshape(tiles, ept)
    a2 = jnp.pad(a2, ((0, 0), (0, chunks * ch - ept)), constant_values=fill)
    a3 = a2.reshape(tiles, chunks, ch)
    a3 = jnp.pad(a3, ((0, 0), (0, cpad - chunks), (0, 0)), constant_values=fill)
    return a3.reshape(tiles * cpad, ch)


def kernel(x, edge_index, W1, b1, W2, b2):
    n = x.shape[0]
    e = edge_index.shape[1]
    src = edge_index[0]
    dst = edge_index[1]

    step = _TILES * _BM
    np_ = ((n + step - 1) // step) * step
    x_p = jnp.pad(x, ((0, np_ - n), (0, 0)))

    spe = _tile_pad_idx(src, 2 * _TILES, _CHUNK, 0)
    dpe = _tile_pad_idx(dst, 2 * _TILES, _CHUNK, np_)
    ones_tab = jnp.ones((np_, 128), _F32)
    deg_cat = _make_agg_edges(np_, e, 128)(ones_tab, spe.reshape(-1), dpe)
    deg0, deg1 = deg_cat[:np_], deg_cat[np_:]

    hs_cat = _tc1(x_p, W1, deg0, deg1)
    sp16 = _tile_pad_idx(src, _TILES, _CHUNK, 0)
    srcp = jnp.concatenate([sp16, sp16 + np_])
    dstp = _tile_pad_idx(dst, _TILES, _CHUNK, np_)
    acc_cat = _make_agg_cols(np_, e, W1.shape[1] // 2)(hs_cat, srcp.reshape(-1), dstp)
    acc0, acc1 = acc_cat[:np_], acc_cat[np_:]

    hs2 = _tc2(acc0, acc1, deg0, deg1, b1.reshape(1, -1), W2)
    p_cat = _make_agg_edges(np_, e, W2.shape[1])(hs2, spe.reshape(-1), dpe)
    p0, p1 = p_cat[:np_], p_cat[np_:]
    return _tc3(p0, p1, hs2, deg0, deg1, b2.reshape(1, -1))[:n]


# speedup vs baseline: 13.5967x; 1.2639x over previous
import functools

import jax
import jax.numpy as jnp
from jax import lax
from jax.experimental import pallas as pl
from jax.experimental.pallas import tpu as pltpu
from jax.experimental.pallas import tpu_sc as plsc

_CHUNK = 128
_F32 = jnp.float32
_TILES = 16
_BM = 640
_NBUF = 2


def _make_deg(n, e):
    tiles = _TILES
    rpt = n // tiles
    ept = e // (2 * tiles)
    chunks = -(-ept // _CHUNK)
    cpad = -(-chunks // 8) * 8
    mesh = plsc.VectorSubcoreMesh(core_axis_name="c", subcore_axis_name="s")

    @functools.partial(
        pl.kernel,
        mesh=mesh,
        out_type=jax.ShapeDtypeStruct((2 * n, 128), _F32),
        scratch_types=[
            pltpu.VMEM((80, _CHUNK), jnp.int32),
            pltpu.VMEM((_CHUNK, 128), _F32),
            pltpu.VMEM_SHARED((n + 8, 128), _F32),
            pltpu.SemaphoreType.DMA,
            pltpu.SemaphoreType.DMA,
        ],
    )
    def deg(dstp, ones, deg_cat, didx_all, ones_v, acc, s0, s1):
        cid = lax.axis_index("c")
        sid = lax.axis_index("s")
        sems = (s0, s1)
        r0 = sid * rpt
        pltpu.sync_copy(ones, ones_v)
        for r in range(rpt // _CHUNK):
            pltpu.sync_copy(ones_v, acc.at[pl.ds(r0 + r * _CHUNK, _CHUNK), :])
        base = pl.multiple_of((cid * tiles + sid) * cpad, 8)
        pltpu.sync_copy(dstp.at[pl.ds(base, 80), :], didx_all)
        plsc.subcore_barrier()

        def s_start(b, c):
            pltpu.async_copy(ones_v, acc.at[didx_all.at[c]], sems[b], add=True)

        def s_wait(b, c):
            pltpu.make_async_copy(ones_v, acc.at[didx_all.at[c]], sems[b]).wait()

        m = chunks // 2

        def body(j, carry):
            for b in range(2):
                s_start(b, 2 * j + b)
            for b in range(2):
                s_wait(b, 2 * j + b)
            return carry

        lax.fori_loop(0, m, body, None)
        for c in range(2 * m, chunks):
            s_start(0, c)
            s_wait(0, c)
        plsc.subcore_barrier()
        ob = pl.multiple_of(cid * n + r0, 8)
        pltpu.sync_copy(acc.at[pl.ds(r0, rpt), :], deg_cat.at[pl.ds(ob, rpt), :])

    return deg


def _agg_body(hs, acc, srcp1, dstp, sidx, didx_all, rbufs, sems,
              sbase, dbase, chunks):
    g = len(rbufs)
    _H = 80

    def do_pair(h, j, nloc):
        for b in range(g):
            cl = g * j + b
            cg = h * _H + cl
            pltpu.sync_copy(srcp1.at[pl.ds((sbase + cg) * _CHUNK, _CHUNK)], sidx[b])
            pltpu.async_copy(hs.at[sidx[b]], rbufs[b], sems[b])
        for b in range(g):
            cl = g * j + b
            pltpu.make_async_copy(hs.at[sidx[b]], rbufs[b], sems[b]).wait()
            pltpu.sync_copy(rbufs[b], acc.at[didx_all.at[cl]], add=True)

    def do_one(h, cl):
        cg = h * _H + cl
        pltpu.sync_copy(srcp1.at[pl.ds((sbase + cg) * _CHUNK, _CHUNK)], sidx[0])
        pltpu.async_copy(hs.at[sidx[0]], rbufs[0], sems[0])
        pltpu.make_async_copy(hs.at[sidx[0]], rbufs[0], sems[0]).wait()
        pltpu.sync_copy(rbufs[0], acc.at[didx_all.at[cl]], add=True)

    for h in range(-(-chunks // _H)):
        nloc = min(_H, chunks - h * _H)
        pltpu.sync_copy(dstp.at[pl.ds(pl.multiple_of(dbase + h * _H, 8), _H), :],
                        didx_all)
        m = nloc // g

        def body(j, carry, h=h, nloc=nloc):
            do_pair(h, j, nloc)
            return carry

        lax.fori_loop(0, m, body, None)
        for cl in range(g * m, nloc):
            do_one(h, cl)


def _agg_scratch(n, dh):
    return [
        pltpu.VMEM((_CHUNK,), jnp.int32),
        pltpu.VMEM((_CHUNK,), jnp.int32),
        pltpu.VMEM((80, _CHUNK), jnp.int32),
    ] + [pltpu.VMEM((_CHUNK, dh), _F32)] * _NBUF + [
        pltpu.VMEM_SHARED((n + 8, dh), _F32),
    ] + [pltpu.SemaphoreType.DMA] * _NBUF


def _make_agg_cols(n, e, dh):
    tiles = _TILES
    rpt = n // tiles
    ept = e // tiles
    chunks = -(-ept // _CHUNK)
    cpad = -(-chunks // 8) * 8
    mesh = plsc.VectorSubcoreMesh(core_axis_name="c", subcore_axis_name="s")

    @functools.partial(
        pl.kernel,
        mesh=mesh,
        out_type=jax.ShapeDtypeStruct((2 * n, dh), _F32),
        scratch_types=_agg_scratch(n, dh),
    )
    def agg(hs_cat, srcp1, dstp, out_cat, si0, si1, didx_all, *rest):
        rbufs, (acc,), sems = rest[:_NBUF], rest[_NBUF:_NBUF + 1], rest[_NBUF + 1:]
        cid = lax.axis_index("c")
        sid = lax.axis_index("s")
        r0 = sid * rpt
        tb = pl.multiple_of(cid * n + r0, 8)
        pltpu.sync_copy(hs_cat.at[pl.ds(tb, rpt), :], acc.at[pl.ds(r0, rpt), :])
        plsc.subcore_barrier()
        sbase = pl.multiple_of(cid * tiles * cpad + sid * cpad, 8)
        dbase = pl.multiple_of(sid * cpad, 8)
        _agg_body(hs_cat, acc, srcp1, dstp, (si0, si1), didx_all, rbufs, sems,
                  sbase, dbase, chunks)
        plsc.subcore_barrier()
        pltpu.sync_copy(acc.at[pl.ds(r0, rpt), :], out_cat.at[pl.ds(tb, rpt), :])

    return agg


def _make_agg_edges(n, e, dh):
    tiles = _TILES
    rpt = n // tiles
    ept = e // (2 * tiles)
    chunks = -(-ept // _CHUNK)
    cpad = -(-chunks // 8) * 8
    mesh = plsc.VectorSubcoreMesh(core_axis_name="c", subcore_axis_name="s")

    @functools.partial(
        pl.kernel,
        mesh=mesh,
        out_type=jax.ShapeDtypeStruct((2 * n, dh), _F32),
        scratch_types=_agg_scratch(n, dh),
    )
    def agg(hs, srcp1, dstp, out_cat, si0, si1, didx_all, *rest):
        rbufs, (acc,), sems = rest[:_NBUF], rest[_NBUF:_NBUF + 1], rest[_NBUF + 1:]
        cid = lax.axis_index("c")
        sid = lax.axis_index("s")
        r0 = sid * rpt
        pltpu.sync_copy(hs.at[pl.ds(r0, rpt), :], acc.at[pl.ds(r0, rpt), :])
        plsc.subcore_barrier()
        base = pl.multiple_of((cid * tiles + sid) * cpad, 8)
        _agg_body(hs, acc, srcp1, dstp, (si0, si1), didx_all, rbufs, sems,
                  base, base, chunks)
        plsc.subcore_barrier()
        ob = pl.multiple_of(cid * n + r0, 8)
        pltpu.sync_copy(acc.at[pl.ds(r0, rpt), :], out_cat.at[pl.ds(ob, rpt), :])

    return agg


def _dinv_of(d0, d1):
    return lax.rsqrt(d0[:, 0:1] + d1[:, 0:1] - 1.0)


def _tc1(x, w1, deg0, deg1):
    n, din = x.shape
    dh2 = w1.shape[1]
    half = dh2 // 2
    npb = n // _BM

    def body(x_ref, w_ref, d0_ref, d1_ref, o_ref):
        dinv = _dinv_of(d0_ref[...], d1_ref[...])
        o_ref[...] = jnp.dot(x_ref[...], w_ref[...], preferred_element_type=_F32) * dinv

    return pl.pallas_call(
        body,
        grid=(2, npb),
        in_specs=[
            pl.BlockSpec((_BM, din), lambda c, i: (i, 0)),
            pl.BlockSpec((din, half), lambda c, i: (0, c)),
            pl.BlockSpec((_BM, 128), lambda c, i: (i, 0)),
            pl.BlockSpec((_BM, 128), lambda c, i: (i, 0)),
        ],
        out_specs=pl.BlockSpec((_BM, half), lambda c, i: (c * npb + i, 0)),
        out_shape=jax.ShapeDtypeStruct((2 * n, half), _F32),
    )(x, w1, deg0, deg1)


def _tc2(acc0, acc1, deg0, deg1, b1, w2):
    n, half1 = acc0.shape
    dh = 2 * half1
    dout = w2.shape[1]

    def body(a0_ref, a1_ref, d0_ref, d1_ref, b_ref, w_ref, o_ref):
        dinv = _dinv_of(d0_ref[...], d1_ref[...])
        a = jnp.concatenate([a0_ref[...], a1_ref[...]], axis=1)
        h1 = jnp.maximum(a * dinv + b_ref[...], 0.0)
        o_ref[...] = jnp.dot(h1, w_ref[...], preferred_element_type=_F32) * dinv

    return pl.pallas_call(
        body,
        grid=(n // _BM,),
        in_specs=[
            pl.BlockSpec((_BM, half1), lambda i: (i, 0)),
            pl.BlockSpec((_BM, half1), lambda i: (i, 0)),
            pl.BlockSpec((_BM, 128), lambda i: (i, 0)),
            pl.BlockSpec((_BM, 128), lambda i: (i, 0)),
            pl.BlockSpec((1, dh), lambda i: (0, 0)),
            pl.BlockSpec((dh, dout), lambda i: (0, 0)),
        ],
        out_specs=pl.BlockSpec((_BM, dout), lambda i: (i, 0)),
        out_shape=jax.ShapeDtypeStruct((n, dout), _F32),
    )(acc0, acc1, deg0, deg1, b1, w2)


def _tc3(p0, p1, hs2, deg0, deg1, b2):
    n, dout = p0.shape

    def body(a0_ref, a1_ref, h_ref, d0_ref, d1_ref, b_ref, o_ref):
        dinv = _dinv_of(d0_ref[...], d1_ref[...])
        a = a0_ref[...] + a1_ref[...] - h_ref[...]
        o_ref[...] = a * dinv + b_ref[...]

    return pl.pallas_call(
        body,
        grid=(n // _BM,),
        in_specs=[
            pl.BlockSpec((_BM, dout), lambda i: (i, 0)),
            pl.BlockSpec((_BM, dout), lambda i: (i, 0)),
            pl.BlockSpec((_BM, dout), lambda i: (i, 0)),
            pl.BlockSpec((_BM, 128), lambda i: (i, 0)),
            pl.BlockSpec((_BM, 128), lambda i: (i, 0)),
            pl.BlockSpec((1, dout), lambda i: (0, 0)),
        ],
        out_specs=pl.BlockSpec((_BM, dout), lambda i: (i, 0)),
        out_shape=jax.ShapeDtypeStruct((n, dout), _F32),
    )(p0, p1, hs2, deg0, deg1, b2)


def _tile_pad_idx(a, tiles, ch, fill):
    ept = a.shape[0] // tiles
    chunks = -(-ept // ch)
    cpad = -(-chunks // 8) * 8
    a2 = a.reshape(tiles, ept)
    a2 = jnp.pad(a2, ((0, 0), (0, chunks * ch - ept)), constant_values=fill)
    a3 = a2.reshape(tiles, chunks, ch)
    a3 = jnp.pad(a3, ((0, 0), (0, cpad - chunks), (0, 0)), constant_values=fill)
    return a3.reshape(tiles * cpad, ch)


def kernel(x, edge_index, W1, b1, W2, b2):
    n = x.shape[0]
    e = edge_index.shape[1]
    src = edge_index[0]
    dst = edge_index[1]

    step = _TILES * _BM
    np_ = ((n + step - 1) // step) * step
    x_p = jnp.pad(x, ((0, np_ - n), (0, 0)))

    spe = _tile_pad_idx(src, 2 * _TILES, _CHUNK, 0)
    dpe = _tile_pad_idx(dst, 2 * _TILES, _CHUNK, np_)
    ones = jnp.ones((_CHUNK, 128), _F32)
    deg_cat = _make_deg(np_, e)(dpe, ones)
    deg0, deg1 = deg_cat[:np_], deg_cat[np_:]

    hs_cat = _tc1(x_p, W1, deg0, deg1)
    sp16 = _tile_pad_idx(src, _TILES, _CHUNK, 0)
    srcp = jnp.concatenate([sp16, sp16 + np_])
    dstp = _tile_pad_idx(dst, _TILES, _CHUNK, np_)
    acc_cat = _make_agg_cols(np_, e, W1.shape[1] // 2)(hs_cat, srcp.reshape(-1), dstp)
    acc0, acc1 = acc_cat[:np_], acc_cat[np_:]

    hs2 = _tc2(acc0, acc1, deg0, deg1, b1.reshape(1, -1), W2)
    p_cat = _make_agg_edges(np_, e, W2.shape[1])(hs2, spe.reshape(-1), dpe)
    p0, p1 = p_cat[:np_], p_cat[np_:]
    return _tc3(p0, p1, hs2, deg0, deg1, b2.reshape(1, -1))[:n]


# speedup vs baseline: 13.8252x; 1.0168x over previous
import functools

import jax
import jax.numpy as jnp
from jax import lax
from jax.experimental import pallas as pl
from jax.experimental.pallas import tpu as pltpu
from jax.experimental.pallas import tpu_sc as plsc

_CHUNK = 128
_F32 = jnp.float32
_TILES = 16
_BM = 640
_NBUF = 2


def _make_deg(n, e):
    tiles = _TILES
    rpt = n // tiles
    ept = e // (2 * tiles)
    chunks = -(-ept // _CHUNK)
    cpad = -(-chunks // 8) * 8
    mesh = plsc.VectorSubcoreMesh(core_axis_name="c", subcore_axis_name="s")

    @functools.partial(
        pl.kernel,
        mesh=mesh,
        out_type=jax.ShapeDtypeStruct((2 * n, 128), _F32),
        scratch_types=[
            pltpu.VMEM((80, _CHUNK), jnp.int32),
            pltpu.VMEM((_CHUNK, 128), _F32),
            pltpu.VMEM_SHARED((n + 8, 128), _F32),
            pltpu.SemaphoreType.DMA,
            pltpu.SemaphoreType.DMA,
        ],
    )
    def deg(dstp, ones, deg_cat, didx_all, ones_v, acc, s0, s1):
        cid = lax.axis_index("c")
        sid = lax.axis_index("s")
        sems = (s0, s1)
        r0 = sid * rpt
        pltpu.sync_copy(ones, ones_v)
        for r in range(rpt // _CHUNK):
            pltpu.sync_copy(ones_v, acc.at[pl.ds(r0 + r * _CHUNK, _CHUNK), :])
        base = pl.multiple_of((cid * tiles + sid) * cpad, 8)
        pltpu.sync_copy(dstp.at[pl.ds(base, 80), :], didx_all)
        plsc.subcore_barrier()

        def s_start(b, c):
            pltpu.async_copy(ones_v, acc.at[didx_all.at[c]], sems[b], add=True)

        def s_wait(b, c):
            pltpu.make_async_copy(ones_v, acc.at[didx_all.at[c]], sems[b]).wait()

        m = chunks // 2

        def body(j, carry):
            for b in range(2):
                s_start(b, 2 * j + b)
            for b in range(2):
                s_wait(b, 2 * j + b)
            return carry

        lax.fori_loop(0, m, body, None)
        for c in range(2 * m, chunks):
            s_start(0, c)
            s_wait(0, c)
        plsc.subcore_barrier()
        ob = pl.multiple_of(cid * n + r0, 8)
        pltpu.sync_copy(acc.at[pl.ds(r0, rpt), :], deg_cat.at[pl.ds(ob, rpt), :])

    return deg


def _agg_body(hs, acc, srcp, dstp, sidx_st, didx_st, rbufs, sems,
              sbase, dbase, chunks):
    g = len(rbufs)
    _H = 40

    def g_start(b, cl):
        pltpu.async_copy(hs.at[sidx_st.at[cl]], rbufs[b], sems[b])

    def g_wait(b, cl):
        pltpu.make_async_copy(hs.at[sidx_st.at[cl]], rbufs[b], sems[b]).wait()

    def scat(b, cl):
        pltpu.sync_copy(rbufs[b], acc.at[didx_st.at[cl]], add=True)

    for h in range(-(-chunks // _H)):
        nloc = min(_H, chunks - h * _H)
        pltpu.sync_copy(srcp.at[pl.ds(pl.multiple_of(sbase + h * _H, 8), _H), :],
                        sidx_st)
        pltpu.sync_copy(dstp.at[pl.ds(pl.multiple_of(dbase + h * _H, 8), _H), :],
                        didx_st)
        m = nloc // g

        def body(j, carry):
            for b in range(g):
                g_start(b, g * j + b)
            for b in range(g):
                cl = g * j + b
                g_wait(b, cl)
                scat(b, cl)
            return carry

        lax.fori_loop(0, m, body, None)
        for cl in range(g * m, nloc):
            g_start(0, cl)
            g_wait(0, cl)
            scat(0, cl)


def _agg_scratch(n, dh):
    return [
        pltpu.VMEM((40, _CHUNK), jnp.int32),
        pltpu.VMEM((40, _CHUNK), jnp.int32),
    ] + [pltpu.VMEM((_CHUNK, dh), _F32)] * _NBUF + [
        pltpu.VMEM_SHARED((n + 8, dh), _F32),
    ] + [pltpu.SemaphoreType.DMA] * _NBUF


def _make_agg_cols(n, e, dh):
    tiles = _TILES
    rpt = n // tiles
    ept = e // tiles
    chunks = -(-ept // _CHUNK)
    cpad = -(-chunks // 8) * 8
    mesh = plsc.VectorSubcoreMesh(core_axis_name="c", subcore_axis_name="s")

    @functools.partial(
        pl.kernel,
        mesh=mesh,
        out_type=jax.ShapeDtypeStruct((2 * n, dh), _F32),
        scratch_types=_agg_scratch(n, dh),
    )
    def agg(hs_cat, srcp, dstp, out_cat, sidx_st, didx_st, *rest):
        rbufs, (acc,), sems = rest[:_NBUF], rest[_NBUF:_NBUF + 1], rest[_NBUF + 1:]
        cid = lax.axis_index("c")
        sid = lax.axis_index("s")
        r0 = sid * rpt
        tb = pl.multiple_of(cid * n + r0, 8)
        pltpu.sync_copy(hs_cat.at[pl.ds(tb, rpt), :], acc.at[pl.ds(r0, rpt), :])
        plsc.subcore_barrier()
        sbase = pl.multiple_of(cid * tiles * cpad + sid * cpad, 8)
        dbase = pl.multiple_of(sid * cpad, 8)
        _agg_body(hs_cat, acc, srcp, dstp, sidx_st, didx_st, rbufs, sems,
                  sbase, dbase, chunks)
        plsc.subcore_barrier()
        pltpu.sync_copy(acc.at[pl.ds(r0, rpt), :], out_cat.at[pl.ds(tb, rpt), :])

    return agg


def _make_agg_edges(n, e, dh):
    tiles = _TILES
    rpt = n // tiles
    ept = e // (2 * tiles)
    chunks = -(-ept // _CHUNK)
    cpad = -(-chunks // 8) * 8
    mesh = plsc.VectorSubcoreMesh(core_axis_name="c", subcore_axis_name="s")

    @functools.partial(
        pl.kernel,
        mesh=mesh,
        out_type=jax.ShapeDtypeStruct((2 * n, dh), _F32),
        scratch_types=_agg_scratch(n, dh),
    )
    def agg(hs, srcp, dstp, out_cat, sidx_st, didx_st, *rest):
        rbufs, (acc,), sems = rest[:_NBUF], rest[_NBUF:_NBUF + 1], rest[_NBUF + 1:]
        cid = lax.axis_index("c")
        sid = lax.axis_index("s")
        r0 = sid * rpt
        pltpu.sync_copy(hs.at[pl.ds(r0, rpt), :], acc.at[pl.ds(r0, rpt), :])
        plsc.subcore_barrier()
        base = pl.multiple_of((cid * tiles + sid) * cpad, 8)
        _agg_body(hs, acc, srcp, dstp, sidx_st, didx_st, rbufs, sems,
                  base, base, chunks)
        plsc.subcore_barrier()
        ob = pl.multiple_of(cid * n + r0, 8)
        pltpu.sync_copy(acc.at[pl.ds(r0, rpt), :], out_cat.at[pl.ds(ob, rpt), :])

    return agg


def _dinv_of(d0, d1):
    return lax.rsqrt(d0[:, 0:1] + d1[:, 0:1] - 1.0)


def _tc1(x, w1, deg0, deg1):
    n, din = x.shape
    dh2 = w1.shape[1]
    half = dh2 // 2
    npb = n // _BM

    def body(x_ref, w_ref, d0_ref, d1_ref, o_ref):
        dinv = _dinv_of(d0_ref[...], d1_ref[...])
        o_ref[...] = jnp.dot(x_ref[...], w_ref[...], preferred_element_type=_F32) * dinv

    return pl.pallas_call(
        body,
        grid=(2, npb),
        in_specs=[
            pl.BlockSpec((_BM, din), lambda c, i: (i, 0)),
            pl.BlockSpec((din, half), lambda c, i: (0, c)),
            pl.BlockSpec((_BM, 128), lambda c, i: (i, 0)),
            pl.BlockSpec((_BM, 128), lambda c, i: (i, 0)),
        ],
        out_specs=pl.BlockSpec((_BM, half), lambda c, i: (c * npb + i, 0)),
        out_shape=jax.ShapeDtypeStruct((2 * n, half), _F32),
    )(x, w1, deg0, deg1)


def _tc2(acc0, acc1, deg0, deg1, b1, w2):
    n, half1 = acc0.shape
    dh = 2 * half1
    dout = w2.shape[1]

    def body(a0_ref, a1_ref, d0_ref, d1_ref, b_ref, w_ref, o_ref):
        dinv = _dinv_of(d0_ref[...], d1_ref[...])
        a = jnp.concatenate([a0_ref[...], a1_ref[...]], axis=1)
        h1 = jnp.maximum(a * dinv + b_ref[...], 0.0)
        o_ref[...] = jnp.dot(h1, w_ref[...], preferred_element_type=_F32) * dinv

    return pl.pallas_call(
        body,
        grid=(n // _BM,),
        in_specs=[
            pl.BlockSpec((_BM, half1), lambda i: (i, 0)),
            pl.BlockSpec((_BM, half1), lambda i: (i, 0)),
            pl.BlockSpec((_BM, 128), lambda i: (i, 0)),
            pl.BlockSpec((_BM, 128), lambda i: (i, 0)),
            pl.BlockSpec((1, dh), lambda i: (0, 0)),
            pl.BlockSpec((dh, dout), lambda i: (0, 0)),
        ],
        out_specs=pl.BlockSpec((_BM, dout), lambda i: (i, 0)),
        out_shape=jax.ShapeDtypeStruct((n, dout), _F32),
    )(acc0, acc1, deg0, deg1, b1, w2)


def _tc3(p0, p1, hs2, deg0, deg1, b2):
    n, dout = p0.shape

    def body(a0_ref, a1_ref, h_ref, d0_ref, d1_ref, b_ref, o_ref):
        dinv = _dinv_of(d0_ref[...], d1_ref[...])
        a = a0_ref[...] + a1_ref[...] - h_ref[...]
        o_ref[...] = a * dinv + b_ref[...]

    return pl.pallas_call(
        body,
        grid=(n // _BM,),
        in_specs=[
            pl.BlockSpec((_BM, dout), lambda i: (i, 0)),
            pl.BlockSpec((_BM, dout), lambda i: (i, 0)),
            pl.BlockSpec((_BM, dout), lambda i: (i, 0)),
            pl.BlockSpec((_BM, 128), lambda i: (i, 0)),
            pl.BlockSpec((_BM, 128), lambda i: (i, 0)),
            pl.BlockSpec((1, dout), lambda i: (0, 0)),
        ],
        out_specs=pl.BlockSpec((_BM, dout), lambda i: (i, 0)),
        out_shape=jax.ShapeDtypeStruct((n, dout), _F32),
    )(p0, p1, hs2, deg0, deg1, b2)


def _tile_pad_idx(a, tiles, ch, fill):
    ept = a.shape[0] // tiles
    chunks = -(-ept // ch)
    cpad = -(-chunks // 8) * 8
    a2 = a.reshape(tiles, ept)
    a2 = jnp.pad(a2, ((0, 0), (0, chunks * ch - ept)), constant_values=fill)
    a3 = a2.reshape(tiles, chunks, ch)
    a3 = jnp.pad(a3, ((0, 0), (0, cpad - chunks), (0, 0)), constant_values=fill)
    return a3.reshape(tiles * cpad, ch)


def kernel(x, edge_index, W1, b1, W2, b2):
    n = x.shape[0]
    e = edge_index.shape[1]
    src = edge_index[0]
    dst = edge_index[1]

    step = _TILES * _BM
    np_ = ((n + step - 1) // step) * step
    x_p = jnp.pad(x, ((0, np_ - n), (0, 0)))

    spe = _tile_pad_idx(src, 2 * _TILES, _CHUNK, 0)
    dpe = _tile_pad_idx(dst, 2 * _TILES, _CHUNK, np_)
    ones = jnp.ones((_CHUNK, 128), _F32)
    deg_cat = _make_deg(np_, e)(dpe, ones)
    deg0, deg1 = deg_cat[:np_], deg_cat[np_:]

    hs_cat = _tc1(x_p, W1, deg0, deg1)
    sp16 = _tile_pad_idx(src, _TILES, _CHUNK, 0)
    srcp = jnp.concatenate([sp16, sp16 + np_])
    dstp = _tile_pad_idx(dst, _TILES, _CHUNK, np_)
    acc_cat = _make_agg_cols(np_, e, W1.shape[1] // 2)(hs_cat, srcp, dstp)
    acc0, acc1 = acc_cat[:np_], acc_cat[np_:]

    hs2 = _tc2(acc0, acc1, deg0, deg1, b1.reshape(1, -1), W2)
    p_cat = _make_agg_edges(np_, e, W2.shape[1])(hs2, spe, dpe)
    p0, p1 = p_cat[:np_], p_cat[np_:]
    return _tc3(p0, p1, hs2, deg0, deg1, b2.reshape(1, -1))[:n]


# speedup vs baseline: 13.9856x; 1.0116x over previous
import functools

import jax
import jax.numpy as jnp
from jax import lax
from jax.experimental import pallas as pl
from jax.experimental.pallas import tpu as pltpu
from jax.experimental.pallas import tpu_sc as plsc

_CHUNK = 128
_F32 = jnp.float32
_TILES = 16
_BM = 640
_NBUF = 2


def _make_deg(n, e):
    tiles = _TILES
    rpt = n // tiles
    ept = e // (2 * tiles)
    chunks = -(-ept // _CHUNK)
    cpad = -(-chunks // 8) * 8
    mesh = plsc.VectorSubcoreMesh(core_axis_name="c", subcore_axis_name="s")

    @functools.partial(
        pl.kernel,
        mesh=mesh,
        out_type=jax.ShapeDtypeStruct((2 * n, 128), _F32),
        scratch_types=[
            pltpu.VMEM((80, _CHUNK), jnp.int32),
            pltpu.VMEM((_CHUNK, 128), _F32),
            pltpu.VMEM_SHARED((n + 8, 128), _F32),
            pltpu.SemaphoreType.DMA,
            pltpu.SemaphoreType.DMA,
        ],
    )
    def deg(dstp, ones, deg_cat, didx_all, ones_v, acc, s0, s1):
        cid = lax.axis_index("c")
        sid = lax.axis_index("s")
        sems = (s0, s1)
        r0 = sid * rpt
        pltpu.sync_copy(ones, ones_v)
        for r in range(rpt // _CHUNK):
            pltpu.sync_copy(ones_v, acc.at[pl.ds(r0 + r * _CHUNK, _CHUNK), :])
        base = pl.multiple_of((cid * tiles + sid) * cpad, 8)
        pltpu.sync_copy(dstp.at[pl.ds(base, 80), :], didx_all)
        plsc.subcore_barrier()

        def s_start(b, c):
            pltpu.async_copy(ones_v, acc.at[didx_all.at[c]], sems[b], add=True)

        def s_wait(b, c):
            pltpu.make_async_copy(ones_v, acc.at[didx_all.at[c]], sems[b]).wait()

        m = chunks // 2

        def body(j, carry):
            for b in range(2):
                s_start(b, 2 * j + b)
            for b in range(2):
                s_wait(b, 2 * j + b)
            return carry

        lax.fori_loop(0, m, body, None)
        for c in range(2 * m, chunks):
            s_start(0, c)
            s_wait(0, c)
        plsc.subcore_barrier()
        ob = pl.multiple_of(cid * n + r0, 8)
        pltpu.sync_copy(acc.at[pl.ds(r0, rpt), :], deg_cat.at[pl.ds(ob, rpt), :])

    return deg


def _agg_body(hs, acc, srcp, dstp, sidx_st, didx_st, rbufs, sems,
              sbase, dbase, chunks):
    g = len(rbufs)
    gsems, ssems = sems[:g], sems[g:]
    _H = 40

    def g_start(b, cl):
        pltpu.async_copy(hs.at[sidx_st.at[cl]], rbufs[b], gsems[b])

    def g_wait(b, cl):
        pltpu.make_async_copy(hs.at[sidx_st.at[cl]], rbufs[b], gsems[b]).wait()

    def s_start(b, cl):
        pltpu.async_copy(rbufs[b], acc.at[didx_st.at[cl]], ssems[b], add=True)

    def s_wait(b, cl):
        pltpu.make_async_copy(rbufs[b], acc.at[didx_st.at[cl]], ssems[b]).wait()

    for h in range(-(-chunks // _H)):
        nloc = min(_H, chunks - h * _H)
        pltpu.sync_copy(srcp.at[pl.ds(pl.multiple_of(sbase + h * _H, 8), _H), :],
                        sidx_st)
        pltpu.sync_copy(dstp.at[pl.ds(pl.multiple_of(dbase + h * _H, 8), _H), :],
                        didx_st)
        m = nloc // g

        def body(j, carry):
            for b in range(g):
                g_start(b, g * j + b)
            for b in range(g):
                cl = g * j + b
                g_wait(b, cl)
                s_start(b, cl)
            for b in range(g):
                s_wait(b, g * j + b)
            return carry

        lax.fori_loop(0, m, body, None)
        for cl in range(g * m, nloc):
            g_start(0, cl)
            g_wait(0, cl)
            s_start(0, cl)
            s_wait(0, cl)


def _agg_scratch(n, dh):
    return [
        pltpu.VMEM((40, _CHUNK), jnp.int32),
        pltpu.VMEM((40, _CHUNK), jnp.int32),
    ] + [pltpu.VMEM((_CHUNK, dh), _F32)] * _NBUF + [
        pltpu.VMEM_SHARED((n + 8, dh), _F32),
    ] + [pltpu.SemaphoreType.DMA] * (2 * _NBUF)


def _make_agg_cols(n, e, dh):
    tiles = _TILES
    rpt = n // tiles
    ept = e // tiles
    chunks = -(-ept // _CHUNK)
    cpad = -(-chunks // 8) * 8
    mesh = plsc.VectorSubcoreMesh(core_axis_name="c", subcore_axis_name="s")

    @functools.partial(
        pl.kernel,
        mesh=mesh,
        out_type=jax.ShapeDtypeStruct((2 * n, dh), _F32),
        scratch_types=_agg_scratch(n, dh),
    )
    def agg(hs_cat, srcp, dstp, out_cat, sidx_st, didx_st, *rest):
        rbufs, (acc,), sems = rest[:_NBUF], rest[_NBUF:_NBUF + 1], rest[_NBUF + 1:]
        cid = lax.axis_index("c")
        sid = lax.axis_index("s")
        r0 = sid * rpt
        tb = pl.multiple_of(cid * n + r0, 8)
        pltpu.sync_copy(hs_cat.at[pl.ds(tb, rpt), :], acc.at[pl.ds(r0, rpt), :])
        plsc.subcore_barrier()
        sbase = pl.multiple_of(cid * tiles * cpad + sid * cpad, 8)
        dbase = pl.multiple_of(sid * cpad, 8)
        _agg_body(hs_cat, acc, srcp, dstp, sidx_st, didx_st, rbufs, sems,
                  sbase, dbase, chunks)
        plsc.subcore_barrier()
        pltpu.sync_copy(acc.at[pl.ds(r0, rpt), :], out_cat.at[pl.ds(tb, rpt), :])

    return agg


def _make_agg_edges(n, e, dh):
    tiles = _TILES
    rpt = n // tiles
    ept = e // (2 * tiles)
    chunks = -(-ept // _CHUNK)
    cpad = -(-chunks // 8) * 8
    mesh = plsc.VectorSubcoreMesh(core_axis_name="c", subcore_axis_name="s")

    @functools.partial(
        pl.kernel,
        mesh=mesh,
        out_type=jax.ShapeDtypeStruct((2 * n, dh), _F32),
        scratch_types=_agg_scratch(n, dh),
    )
    def agg(hs, srcp, dstp, out_cat, sidx_st, didx_st, *rest):
        rbufs, (acc,), sems = rest[:_NBUF], rest[_NBUF:_NBUF + 1], rest[_NBUF + 1:]
        cid = lax.axis_index("c")
        sid = lax.axis_index("s")
        r0 = sid * rpt
        pltpu.sync_copy(hs.at[pl.ds(r0, rpt), :], acc.at[pl.ds(r0, rpt), :])
        plsc.subcore_barrier()
        base = pl.multiple_of((cid * tiles + sid) * cpad, 8)
        _agg_body(hs, acc, srcp, dstp, sidx_st, didx_st, rbufs, sems,
                  base, base, chunks)
        plsc.subcore_barrier()
        ob = pl.multiple_of(cid * n + r0, 8)
        pltpu.sync_copy(acc.at[pl.ds(r0, rpt), :], out_cat.at[pl.ds(ob, rpt), :])

    return agg


def _dinv_of(d0, d1):
    return lax.rsqrt(d0[:, 0:1] + d1[:, 0:1] - 1.0)


def _tc1(x, w1, deg0, deg1):
    n, din = x.shape
    dh2 = w1.shape[1]
    half = dh2 // 2
    npb = n // _BM

    def body(x_ref, w_ref, d0_ref, d1_ref, o_ref):
        dinv = _dinv_of(d0_ref[...], d1_ref[...])
        o_ref[...] = jnp.dot(x_ref[...], w_ref[...], preferred_element_type=_F32) * dinv

    return pl.pallas_call(
        body,
        grid=(2, npb),
        in_specs=[
            pl.BlockSpec((_BM, din), lambda c, i: (i, 0)),
            pl.BlockSpec((din, half), lambda c, i: (0, c)),
            pl.BlockSpec((_BM, 128), lambda c, i: (i, 0)),
            pl.BlockSpec((_BM, 128), lambda c, i: (i, 0)),
        ],
        out_specs=pl.BlockSpec((_BM, half), lambda c, i: (c * npb + i, 0)),
        out_shape=jax.ShapeDtypeStruct((2 * n, half), _F32),
    )(x, w1, deg0, deg1)


def _tc2(acc0, acc1, deg0, deg1, b1, w2):
    n, half1 = acc0.shape
    dh = 2 * half1
    dout = w2.shape[1]

    def body(a0_ref, a1_ref, d0_ref, d1_ref, b_ref, w_ref, o_ref):
        dinv = _dinv_of(d0_ref[...], d1_ref[...])
        a = jnp.concatenate([a0_ref[...], a1_ref[...]], axis=1)
        h1 = jnp.maximum(a * dinv + b_ref[...], 0.0)
        o_ref[...] = jnp.dot(h1, w_ref[...], preferred_element_type=_F32) * dinv

    return pl.pallas_call(
        body,
        grid=(n // _BM,),
        in_specs=[
            pl.BlockSpec((_BM, half1), lambda i: (i, 0)),
            pl.BlockSpec((_BM, half1), lambda i: (i, 0)),
            pl.BlockSpec((_BM, 128), lambda i: (i, 0)),
            pl.BlockSpec((_BM, 128), lambda i: (i, 0)),
            pl.BlockSpec((1, dh), lambda i: (0, 0)),
            pl.BlockSpec((dh, dout), lambda i: (0, 0)),
        ],
        out_specs=pl.BlockSpec((_BM, dout), lambda i: (i, 0)),
        out_shape=jax.ShapeDtypeStruct((n, dout), _F32),
    )(acc0, acc1, deg0, deg1, b1, w2)


def _tc3(p0, p1, hs2, deg0, deg1, b2):
    n, dout = p0.shape

    def body(a0_ref, a1_ref, h_ref, d0_ref, d1_ref, b_ref, o_ref):
        dinv = _dinv_of(d0_ref[...], d1_ref[...])
        a = a0_ref[...] + a1_ref[...] - h_ref[...]
        o_ref[...] = a * dinv + b_ref[...]

    return pl.pallas_call(
        body,
        grid=(n // _BM,),
        in_specs=[
            pl.BlockSpec((_BM, dout), lambda i: (i, 0)),
            pl.BlockSpec((_BM, dout), lambda i: (i, 0)),
            pl.BlockSpec((_BM, dout), lambda i: (i, 0)),
            pl.BlockSpec((_BM, 128), lambda i: (i, 0)),
            pl.BlockSpec((_BM, 128), lambda i: (i, 0)),
            pl.BlockSpec((1, dout), lambda i: (0, 0)),
        ],
        out_specs=pl.BlockSpec((_BM, dout), lambda i: (i, 0)),
        out_shape=jax.ShapeDtypeStruct((n, dout), _F32),
    )(p0, p1, hs2, deg0, deg1, b2)


def _tile_pad_idx(a, tiles, ch, fill):
    ept = a.shape[0] // tiles
    chunks = -(-ept // ch)
    cpad = -(-chunks // 8) * 8
    a2 = a.reshape(tiles, ept)
    a2 = jnp.pad(a2, ((0, 0), (0, chunks * ch - ept)), constant_values=fill)
    a3 = a2.reshape(tiles, chunks, ch)
    a3 = jnp.pad(a3, ((0, 0), (0, cpad - chunks), (0, 0)), constant_values=fill)
    return a3.reshape(tiles * cpad, ch)


def kernel(x, edge_index, W1, b1, W2, b2):
    n = x.shape[0]
    e = edge_index.shape[1]
    src = edge_index[0]
    dst = edge_index[1]

    step = _TILES * _BM
    np_ = ((n + step - 1) // step) * step
    x_p = jnp.pad(x, ((0, np_ - n), (0, 0)))

    spe = _tile_pad_idx(src, 2 * _TILES, _CHUNK, 0)
    dpe = _tile_pad_idx(dst, 2 * _TILES, _CHUNK, np_)
    ones = jnp.ones((_CHUNK, 128), _F32)
    deg_cat = _make_deg(np_, e)(dpe, ones)
    deg0, deg1 = deg_cat[:np_], deg_cat[np_:]

    hs_cat = _tc1(x_p, W1, deg0, deg1)
    sp16 = _tile_pad_idx(src, _TILES, _CHUNK, 0)
    srcp = jnp.concatenate([sp16, sp16 + np_])
    dstp = _tile_pad_idx(dst, _TILES, _CHUNK, np_)
    acc_cat = _make_agg_cols(np_, e, W1.shape[1] // 2)(hs_cat, srcp, dstp)
    acc0, acc1 = acc_cat[:np_], acc_cat[np_:]

    hs2 = _tc2(acc0, acc1, deg0, deg1, b1.reshape(1, -1), W2)
    p_cat = _make_agg_edges(np_, e, W2.shape[1])(hs2, spe, dpe)
    p0, p1 = p_cat[:np_], p_cat[np_:]
    return _tc3(p0, p1, hs2, deg0, deg1, b2.reshape(1, -1))[:n]


# speedup vs baseline: 14.9131x; 1.0663x over previous
import functools

import jax
import jax.numpy as jnp
from jax import lax
from jax.experimental import pallas as pl
from jax.experimental.pallas import tpu as pltpu
from jax.experimental.pallas import tpu_sc as plsc

_CHUNK = 128
_F32 = jnp.float32
_TILES = 16
_BM = 640
_NBUF = 2


def _make_deg(n, e):
    tiles = _TILES
    rpt = n // tiles
    ept = e // (2 * tiles)
    chunks = -(-ept // _CHUNK)
    cpad = -(-chunks // 8) * 8
    mesh = plsc.VectorSubcoreMesh(core_axis_name="c", subcore_axis_name="s")

    @functools.partial(
        pl.kernel,
        mesh=mesh,
        out_type=jax.ShapeDtypeStruct((2 * n, 128), _F32),
        scratch_types=[
            pltpu.VMEM((80, _CHUNK), jnp.int32),
            pltpu.VMEM((_CHUNK, 128), _F32),
            pltpu.VMEM_SHARED((n + 8, 128), _F32),
            pltpu.SemaphoreType.DMA,
            pltpu.SemaphoreType.DMA,
        ],
    )
    def deg(dstp, ones, deg_cat, didx_all, ones_v, acc, s0, s1):
        cid = lax.axis_index("c")
        sid = lax.axis_index("s")
        sems = (s0, s1)
        r0 = sid * rpt
        pltpu.sync_copy(ones, ones_v)
        for r in range(rpt // _CHUNK):
            pltpu.sync_copy(ones_v, acc.at[pl.ds(r0 + r * _CHUNK, _CHUNK), :])
        base = pl.multiple_of((cid * tiles + sid) * cpad, 8)
        pltpu.sync_copy(dstp.at[pl.ds(base, 80), :], didx_all)
        plsc.subcore_barrier()

        def s_start(b, c):
            pltpu.async_copy(ones_v, acc.at[didx_all.at[c]], sems[b], add=True)

        def s_wait(b, c):
            pltpu.make_async_copy(ones_v, acc.at[didx_all.at[c]], sems[b]).wait()

        m = chunks // 2

        def body(j, carry):
            for b in range(2):
                s_start(b, 2 * j + b)
            for b in range(2):
                s_wait(b, 2 * j + b)
            return carry

        lax.fori_loop(0, m, body, None)
        for c in range(2 * m, chunks):
            s_start(0, c)
            s_wait(0, c)
        plsc.subcore_barrier()
        ob = pl.multiple_of(cid * n + r0, 8)
        pltpu.sync_copy(acc.at[pl.ds(r0, rpt), :], deg_cat.at[pl.ds(ob, rpt), :])

    return deg


def _agg_body(hs, acc, srcp, dstp, sidx_st, didx_st, rbufs, sems,
              sbase, dbase, chunks):
    g = len(rbufs)
    gsems, ssems = sems[:g], sems[g:]
    _H = 40

    def g_start(b, cl):
        pltpu.async_copy(hs.at[sidx_st.at[cl]], rbufs[b], gsems[b])

    def g_wait(b, cl):
        pltpu.make_async_copy(hs.at[sidx_st.at[cl]], rbufs[b], gsems[b]).wait()

    def s_start(b, cl):
        pltpu.async_copy(rbufs[b], acc.at[didx_st.at[cl]], ssems[b], add=True)

    def s_wait(b, cl):
        pltpu.make_async_copy(rbufs[b], acc.at[didx_st.at[cl]], ssems[b]).wait()

    for h in range(-(-chunks // _H)):
        nloc = min(_H, chunks - h * _H)
        pltpu.sync_copy(srcp.at[pl.ds(pl.multiple_of(sbase + h * _H, 8), _H), :],
                        sidx_st)
        pltpu.sync_copy(dstp.at[pl.ds(pl.multiple_of(dbase + h * _H, 8), _H), :],
                        didx_st)
        m = nloc // g

        def body(j, carry):
            for b in range(g):
                g_start(b, g * j + b)
            for b in range(g):
                cl = g * j + b
                g_wait(b, cl)
                s_start(b, cl)
            for b in range(g):
                s_wait(b, g * j + b)
            return carry

        lax.fori_loop(0, m, body, None)
        for cl in range(g * m, nloc):
            g_start(0, cl)
            g_wait(0, cl)
            s_start(0, cl)
            s_wait(0, cl)


def _agg_scratch(n, dh):
    return [
        pltpu.VMEM((40, _CHUNK), jnp.int32),
        pltpu.VMEM((40, _CHUNK), jnp.int32),
    ] + [pltpu.VMEM((_CHUNK, dh), _F32)] * _NBUF + [
        pltpu.VMEM_SHARED((n + 8, dh), _F32),
    ] + [pltpu.SemaphoreType.DMA] * (2 * _NBUF)


def _make_agg_cols(n, e, dh):
    tiles = _TILES
    rpt = n // tiles
    ept = e // tiles
    chunks = -(-ept // _CHUNK)
    cpad = -(-chunks // 8) * 8
    mesh = plsc.VectorSubcoreMesh(core_axis_name="c", subcore_axis_name="s")

    @functools.partial(
        pl.kernel,
        mesh=mesh,
        out_type=jax.ShapeDtypeStruct((2 * n, dh), _F32),
        scratch_types=_agg_scratch(n, dh),
    )
    def agg(hs_cat, srcp, dstp, out_cat, sidx_st, didx_st, *rest):
        rbufs, (acc,), sems = rest[:_NBUF], rest[_NBUF:_NBUF + 1], rest[_NBUF + 1:]
        cid = lax.axis_index("c")
        sid = lax.axis_index("s")
        r0 = sid * rpt
        tb = pl.multiple_of(cid * n + r0, 8)
        pltpu.sync_copy(hs_cat.at[pl.ds(tb, rpt), :], acc.at[pl.ds(r0, rpt), :])
        plsc.subcore_barrier()
        sbase = pl.multiple_of(cid * tiles * cpad + sid * cpad, 8)
        dbase = pl.multiple_of(sid * cpad, 8)
        _agg_body(hs_cat, acc, srcp, dstp, sidx_st, didx_st, rbufs, sems,
                  sbase, dbase, chunks)
        plsc.subcore_barrier()
        pltpu.sync_copy(acc.at[pl.ds(r0, rpt), :], out_cat.at[pl.ds(tb, rpt), :])

    return agg


def _make_agg_edges(n, e, dh):
    tiles = _TILES
    rpt = n // tiles
    ept = e // (2 * tiles)
    chunks = -(-ept // _CHUNK)
    cpad = -(-chunks // 8) * 8
    mesh = plsc.VectorSubcoreMesh(core_axis_name="c", subcore_axis_name="s")

    @functools.partial(
        pl.kernel,
        mesh=mesh,
        out_type=jax.ShapeDtypeStruct((2 * n, dh), _F32),
        scratch_types=_agg_scratch(n, dh),
    )
    def agg(hs, srcp, dstp, out_cat, sidx_st, didx_st, *rest):
        rbufs, (acc,), sems = rest[:_NBUF], rest[_NBUF:_NBUF + 1], rest[_NBUF + 1:]
        cid = lax.axis_index("c")
        sid = lax.axis_index("s")
        r0 = sid * rpt
        tb = pl.multiple_of(cid * n + r0, 8)
        pltpu.sync_copy(hs.at[pl.ds(tb, rpt), :], acc.at[pl.ds(r0, rpt), :])
        plsc.subcore_barrier()
        base = pl.multiple_of((cid * tiles + sid) * cpad, 8)
        _agg_body(hs, acc, srcp, dstp, sidx_st, didx_st, rbufs, sems,
                  base, base, chunks)
        plsc.subcore_barrier()
        ob = pl.multiple_of(cid * n + r0, 8)
        pltpu.sync_copy(acc.at[pl.ds(r0, rpt), :], out_cat.at[pl.ds(ob, rpt), :])

    return agg


def _dinv_of(d0, d1):
    return lax.rsqrt(d0[:, 0:1] + d1[:, 0:1] - 1.0)


def _tc1(x, w1, deg0, deg1):
    n, din = x.shape
    dh2 = w1.shape[1]
    half = dh2 // 2
    npb = n // _BM

    def body(x_ref, w_ref, d0_ref, d1_ref, o_ref):
        dinv = _dinv_of(d0_ref[...], d1_ref[...])
        o_ref[...] = jnp.dot(x_ref[...], w_ref[...], preferred_element_type=_F32) * dinv

    return pl.pallas_call(
        body,
        grid=(2, npb),
        in_specs=[
            pl.BlockSpec((_BM, din), lambda c, i: (i, 0)),
            pl.BlockSpec((din, half), lambda c, i: (0, c)),
            pl.BlockSpec((_BM, 128), lambda c, i: (i, 0)),
            pl.BlockSpec((_BM, 128), lambda c, i: (i, 0)),
        ],
        out_specs=pl.BlockSpec((_BM, half), lambda c, i: (c * npb + i, 0)),
        out_shape=jax.ShapeDtypeStruct((2 * n, half), _F32),
    )(x, w1, deg0, deg1)


def _tc2(acc0, acc1, deg0, deg1, b1, w2):
    n, half1 = acc0.shape
    dh = 2 * half1
    dout = w2.shape[1]

    npb = n // _BM

    def body(a0_ref, a1_ref, d0_ref, d1_ref, b_ref, w_ref, o_ref):
        dinv = _dinv_of(d0_ref[...], d1_ref[...])
        a = jnp.concatenate([a0_ref[...], a1_ref[...]], axis=1)
        h1 = jnp.maximum(a * dinv + b_ref[...], 0.0)
        o_ref[...] = jnp.dot(h1, w_ref[...], preferred_element_type=_F32) * dinv

    return pl.pallas_call(
        body,
        grid=(2, npb),
        in_specs=[
            pl.BlockSpec((_BM, half1), lambda c, i: (i, 0)),
            pl.BlockSpec((_BM, half1), lambda c, i: (i, 0)),
            pl.BlockSpec((_BM, 128), lambda c, i: (i, 0)),
            pl.BlockSpec((_BM, 128), lambda c, i: (i, 0)),
            pl.BlockSpec((1, dh), lambda c, i: (0, 0)),
            pl.BlockSpec((dh, dout), lambda c, i: (0, 0)),
        ],
        out_specs=pl.BlockSpec((_BM, dout), lambda c, i: (c * npb + i, 0)),
        out_shape=jax.ShapeDtypeStruct((2 * n, dout), _F32),
    )(acc0, acc1, deg0, deg1, b1, w2)


def _tc3(p0, p1, hs2, deg0, deg1, b2):
    n, dout = p0.shape

    def body(a0_ref, a1_ref, h_ref, d0_ref, d1_ref, b_ref, o_ref):
        dinv = _dinv_of(d0_ref[...], d1_ref[...])
        a = a0_ref[...] + a1_ref[...] - h_ref[...]
        o_ref[...] = a * dinv + b_ref[...]

    return pl.pallas_call(
        body,
        grid=(n // _BM,),
        in_specs=[
            pl.BlockSpec((_BM, dout), lambda i: (i, 0)),
            pl.BlockSpec((_BM, dout), lambda i: (i, 0)),
            pl.BlockSpec((_BM, dout), lambda i: (i, 0)),
            pl.BlockSpec((_BM, 128), lambda i: (i, 0)),
            pl.BlockSpec((_BM, 128), lambda i: (i, 0)),
            pl.BlockSpec((1, dout), lambda i: (0, 0)),
        ],
        out_specs=pl.BlockSpec((_BM, dout), lambda i: (i, 0)),
        out_shape=jax.ShapeDtypeStruct((n, dout), _F32),
    )(p0, p1, hs2, deg0, deg1, b2)


def _tile_pad_idx(a, tiles, ch, fill):
    ept = a.shape[0] // tiles
    chunks = -(-ept // ch)
    cpad = -(-chunks // 8) * 8
    a2 = a.reshape(tiles, ept)
    a2 = jnp.pad(a2, ((0, 0), (0, chunks * ch - ept)), constant_values=fill)
    a3 = a2.reshape(tiles, chunks, ch)
    a3 = jnp.pad(a3, ((0, 0), (0, cpad - chunks), (0, 0)), constant_values=fill)
    return a3.reshape(tiles * cpad, ch)


def kernel(x, edge_index, W1, b1, W2, b2):
    n = x.shape[0]
    e = edge_index.shape[1]
    src = edge_index[0]
    dst = edge_index[1]

    step = _TILES * _BM
    np_ = ((n + step - 1) // step) * step
    x_p = jnp.pad(x, ((0, np_ - n), (0, 0)))

    spe = _tile_pad_idx(src, 2 * _TILES, _CHUNK, 0)
    dpe = _tile_pad_idx(dst, 2 * _TILES, _CHUNK, np_)
    ones = jnp.ones((_CHUNK, 128), _F32)
    deg_cat = _make_deg(np_, e)(dpe, ones)
    deg0, deg1 = deg_cat[:np_], deg_cat[np_:]

    hs_cat = _tc1(x_p, W1, deg0, deg1)
    sp16 = _tile_pad_idx(src, _TILES, _CHUNK, 0)
    srcp = jnp.concatenate([sp16, sp16 + np_])
    dstp = _tile_pad_idx(dst, _TILES, _CHUNK, np_)
    acc_cat = _make_agg_cols(np_, e, W1.shape[1] // 2)(hs_cat, srcp, dstp)
    acc0, acc1 = acc_cat[:np_], acc_cat[np_:]

    hs2_cat = _tc2(acc0, acc1, deg0, deg1, b1.reshape(1, -1), W2)
    half_rows = spe.shape[0] // 2
    spe2 = jnp.concatenate([spe[:half_rows], spe[half_rows:] + np_])
    p_cat = _make_agg_edges(np_, e, W2.shape[1])(hs2_cat, spe2, dpe)
    p0, p1 = p_cat[:np_], p_cat[np_:]
    return _tc3(p0, p1, hs2_cat[:np_], deg0, deg1, b2.reshape(1, -1))[:n]


# speedup vs baseline: 14.9131x; 1.0000x over previous
import functools

import jax
import jax.numpy as jnp
from jax import lax
from jax.experimental import pallas as pl
from jax.experimental.pallas import tpu as pltpu
from jax.experimental.pallas import tpu_sc as plsc

_CHUNK = 128
_F32 = jnp.float32
_TILES = 16
_BM = 640
_NBUF = 2


def _make_deg(n, e):
    tiles = _TILES
    rpt = n // tiles
    ept = e // (2 * tiles)
    chunks = -(-ept // _CHUNK)
    cpad = -(-chunks // 8) * 8
    mesh = plsc.VectorSubcoreMesh(core_axis_name="c", subcore_axis_name="s")

    @functools.partial(
        pl.kernel,
        mesh=mesh,
        out_type=jax.ShapeDtypeStruct((2 * n, 128), _F32),
        scratch_types=[
            pltpu.VMEM((80, _CHUNK), jnp.int32),
            pltpu.VMEM((_CHUNK, 128), _F32),
            pltpu.VMEM_SHARED((n + 8, 128), _F32),
            pltpu.SemaphoreType.DMA,
            pltpu.SemaphoreType.DMA,
        ],
    )
    def deg(dstp, ones, deg_cat, didx_all, ones_v, acc, s0, s1):
        cid = lax.axis_index("c")
        sid = lax.axis_index("s")
        sems = (s0, s1)
        r0 = sid * rpt
        pltpu.sync_copy(ones, ones_v)
        for r in range(rpt // _CHUNK):
            pltpu.sync_copy(ones_v, acc.at[pl.ds(r0 + r * _CHUNK, _CHUNK), :])
        base = pl.multiple_of((cid * tiles + sid) * cpad, 8)
        pltpu.sync_copy(dstp.at[pl.ds(base, 80), :], didx_all)
        plsc.subcore_barrier()

        def s_start(b, c):
            pltpu.async_copy(ones_v, acc.at[didx_all.at[c]], sems[b], add=True)

        def s_wait(b, c):
            pltpu.make_async_copy(ones_v, acc.at[didx_all.at[c]], sems[b]).wait()

        m = chunks // 2

        def body(j, carry):
            for b in range(2):
                s_start(b, 2 * j + b)
            for b in range(2):
                s_wait(b, 2 * j + b)
            return carry

        lax.fori_loop(0, m, body, None)
        for c in range(2 * m, chunks):
            s_start(0, c)
            s_wait(0, c)
        plsc.subcore_barrier()
        ob = pl.multiple_of(cid * n + r0, 8)
        pltpu.sync_copy(acc.at[pl.ds(r0, rpt), :], deg_cat.at[pl.ds(ob, rpt), :])

    return deg


def _agg_body(hs, acc, srcp, dstp, sidx_st, didx_st, rbufs, sems,
              sbase, dbase, chunks):
    g = len(rbufs)
    gsems, ssems = sems[:g], sems[g:]
    _H = 40

    def g_start(b, cl):
        pltpu.async_copy(hs.at[sidx_st.at[cl]], rbufs[b], gsems[b])

    def g_wait(b, cl):
        pltpu.make_async_copy(hs.at[sidx_st.at[cl]], rbufs[b], gsems[b]).wait()

    def s_start(b, cl):
        pltpu.async_copy(rbufs[b], acc.at[didx_st.at[cl]], ssems[b], add=True)

    def s_wait(b, cl):
        pltpu.make_async_copy(rbufs[b], acc.at[didx_st.at[cl]], ssems[b]).wait()

    for h in range(-(-chunks // _H)):
        nloc = min(_H, chunks - h * _H)
        pltpu.sync_copy(srcp.at[pl.ds(pl.multiple_of(sbase + h * _H, 8), _H), :],
                        sidx_st)
        pltpu.sync_copy(dstp.at[pl.ds(pl.multiple_of(dbase + h * _H, 8), _H), :],
                        didx_st)
        m = nloc // g

        def body(j, carry):
            for b in range(g):
                g_start(b, g * j + b)
            for b in range(g):
                cl = g * j + b
                g_wait(b, cl)
                s_start(b, cl)
            for b in range(g):
                s_wait(b, g * j + b)
            return carry

        lax.fori_loop(0, m, body, None)
        for cl in range(g * m, nloc):
            g_start(0, cl)
            g_wait(0, cl)
            s_start(0, cl)
            s_wait(0, cl)


def _agg_scratch(n, dh):
    return [
        pltpu.VMEM((40, _CHUNK), jnp.int32),
        pltpu.VMEM((40, _CHUNK), jnp.int32),
    ] + [pltpu.VMEM((_CHUNK, dh), _F32)] * _NBUF + [
        pltpu.VMEM_SHARED((n + 8, dh), _F32),
    ] + [pltpu.SemaphoreType.DMA] * (2 * _NBUF)


def _make_agg_cols(n, e, dh):
    tiles = _TILES
    rpt = n // tiles
    ept = e // tiles
    chunks = -(-ept // _CHUNK)
    cpad = -(-chunks // 8) * 8
    mesh = plsc.VectorSubcoreMesh(core_axis_name="c", subcore_axis_name="s")

    @functools.partial(
        pl.kernel,
        mesh=mesh,
        out_type=jax.ShapeDtypeStruct((2 * n, dh), _F32),
        scratch_types=_agg_scratch(n, dh),
    )
    def agg(hs_cat, srcp, dstp, out_cat, sidx_st, didx_st, *rest):
        rbufs, (acc,), sems = rest[:_NBUF], rest[_NBUF:_NBUF + 1], rest[_NBUF + 1:]
        cid = lax.axis_index("c")
        sid = lax.axis_index("s")
        r0 = sid * rpt
        tb = pl.multiple_of(cid * n + r0, 8)
        pltpu.sync_copy(hs_cat.at[pl.ds(tb, rpt), :], acc.at[pl.ds(r0, rpt), :])
        plsc.subcore_barrier()
        sbase = pl.multiple_of(cid * tiles * cpad + sid * cpad, 8)
        dbase = pl.multiple_of(sid * cpad, 8)
        _agg_body(hs_cat, acc, srcp, dstp, sidx_st, didx_st, rbufs, sems,
                  sbase, dbase, chunks)
        plsc.subcore_barrier()
        pltpu.sync_copy(acc.at[pl.ds(r0, rpt), :], out_cat.at[pl.ds(tb, rpt), :])

    return agg


def _make_agg_edges(n, e, dh):
    tiles = _TILES
    rpt = n // tiles
    ept = e // (2 * tiles)
    chunks = -(-ept // _CHUNK)
    cpad = -(-chunks // 8) * 8
    mesh = plsc.VectorSubcoreMesh(core_axis_name="c", subcore_axis_name="s")

    @functools.partial(
        pl.kernel,
        mesh=mesh,
        out_type=jax.ShapeDtypeStruct((2 * n, dh), _F32),
        scratch_types=_agg_scratch(n, dh),
    )
    def agg(hs, srcp, dstp, out_cat, sidx_st, didx_st, *rest):
        rbufs, (acc,), sems = rest[:_NBUF], rest[_NBUF:_NBUF + 1], rest[_NBUF + 1:]
        cid = lax.axis_index("c")
        sid = lax.axis_index("s")
        r0 = sid * rpt
        tb = pl.multiple_of(cid * n + r0, 8)
        pltpu.sync_copy(hs.at[pl.ds(tb, rpt), :], acc.at[pl.ds(r0, rpt), :])
        plsc.subcore_barrier()
        base = pl.multiple_of((cid * tiles + sid) * cpad, 8)
        _agg_body(hs, acc, srcp, dstp, sidx_st, didx_st, rbufs, sems,
                  base, base, chunks)
        plsc.subcore_barrier()
        ob = pl.multiple_of(cid * n + r0, 8)
        pltpu.sync_copy(acc.at[pl.ds(r0, rpt), :], out_cat.at[pl.ds(ob, rpt), :])

    return agg


def _dinv_of(d0, d1):
    return lax.rsqrt(d0[:, 0:1] + d1[:, 0:1] - 1.0)


def _tc1(x, w1, deg0, deg1):
    n, din = x.shape
    dh2 = w1.shape[1]
    half = dh2 // 2
    npb = n // _BM

    def body(x_ref, w_ref, d0_ref, d1_ref, o_ref, dv_ref):
        dinv = _dinv_of(d0_ref[...], d1_ref[...])
        o_ref[...] = jnp.dot(x_ref[...], w_ref[...], preferred_element_type=_F32) * dinv
        dv_ref[...] = jnp.broadcast_to(dinv, (dinv.shape[0], 16))

    return pl.pallas_call(
        body,
        grid=(2, npb),
        in_specs=[
            pl.BlockSpec((_BM, din), lambda c, i: (i, 0)),
            pl.BlockSpec((din, half), lambda c, i: (0, c)),
            pl.BlockSpec((_BM, 128), lambda c, i: (i, 0)),
            pl.BlockSpec((_BM, 128), lambda c, i: (i, 0)),
        ],
        out_specs=[
            pl.BlockSpec((_BM, half), lambda c, i: (c * npb + i, 0)),
            pl.BlockSpec((_BM, 16), lambda c, i: (i, 0)),
        ],
        out_shape=[
            jax.ShapeDtypeStruct((2 * n, half), _F32),
            jax.ShapeDtypeStruct((n, 16), _F32),
        ],
    )(x, w1, deg0, deg1)


def _tc2(acc0, acc1, dinv16, b1, w2):
    n, half1 = acc0.shape
    dh = 2 * half1
    dout = w2.shape[1]

    npb = n // _BM

    def body(a0_ref, a1_ref, dv_ref, b_ref, w_ref, o_ref):
        dinv = dv_ref[:, 0:1]
        a = jnp.concatenate([a0_ref[...], a1_ref[...]], axis=1)
        h1 = jnp.maximum(a * dinv + b_ref[...], 0.0)
        o_ref[...] = jnp.dot(h1, w_ref[...], preferred_element_type=_F32) * dinv

    return pl.pallas_call(
        body,
        grid=(2, npb),
        in_specs=[
            pl.BlockSpec((_BM, half1), lambda c, i: (i, 0)),
            pl.BlockSpec((_BM, half1), lambda c, i: (i, 0)),
            pl.BlockSpec((_BM, 16), lambda c, i: (i, 0)),
            pl.BlockSpec((1, dh), lambda c, i: (0, 0)),
            pl.BlockSpec((dh, dout), lambda c, i: (0, 0)),
        ],
        out_specs=pl.BlockSpec((_BM, dout), lambda c, i: (c * npb + i, 0)),
        out_shape=jax.ShapeDtypeStruct((2 * n, dout), _F32),
    )(acc0, acc1, dinv16, b1, w2)


def _tc3(p0, p1, hs2, dinv16, b2):
    n, dout = p0.shape

    def body(a0_ref, a1_ref, h_ref, dv_ref, b_ref, o_ref):
        dinv = dv_ref[:, 0:1]
        a = a0_ref[...] + a1_ref[...] - h_ref[...]
        o_ref[...] = a * dinv + b_ref[...]

    return pl.pallas_call(
        body,
        grid=(n // _BM,),
        in_specs=[
            pl.BlockSpec((_BM, dout), lambda i: (i, 0)),
            pl.BlockSpec((_BM, dout), lambda i: (i, 0)),
            pl.BlockSpec((_BM, dout), lambda i: (i, 0)),
            pl.BlockSpec((_BM, 16), lambda i: (i, 0)),
            pl.BlockSpec((1, dout), lambda i: (0, 0)),
        ],
        out_specs=pl.BlockSpec((_BM, dout), lambda i: (i, 0)),
        out_shape=jax.ShapeDtypeStruct((n, dout), _F32),
    )(p0, p1, hs2, dinv16, b2)


def _tile_pad_idx(a, tiles, ch, fill):
    ept = a.shape[0] // tiles
    chunks = -(-ept // ch)
    cpad = -(-chunks // 8) * 8
    a2 = a.reshape(tiles, ept)
    a2 = jnp.pad(a2, ((0, 0), (0, chunks * ch - ept)), constant_values=fill)
    a3 = a2.reshape(tiles, chunks, ch)
    a3 = jnp.pad(a3, ((0, 0), (0, cpad - chunks), (0, 0)), constant_values=fill)
    return a3.reshape(tiles * cpad, ch)


def kernel(x, edge_index, W1, b1, W2, b2):
    n = x.shape[0]
    e = edge_index.shape[1]
    src = edge_index[0]
    dst = edge_index[1]

    step = _TILES * _BM
    np_ = ((n + step - 1) // step) * step
    x_p = jnp.pad(x, ((0, np_ - n), (0, 0)))

    spe = _tile_pad_idx(src, 2 * _TILES, _CHUNK, 0)
    dpe = _tile_pad_idx(dst, 2 * _TILES, _CHUNK, np_)
    ones = jnp.ones((_CHUNK, 128), _F32)
    deg_cat = _make_deg(np_, e)(dpe, ones)
    deg0, deg1 = deg_cat[:np_], deg_cat[np_:]

    hs_cat, dinv16 = _tc1(x_p, W1, deg0, deg1)
    sp16 = _tile_pad_idx(src, _TILES, _CHUNK, 0)
    srcp = jnp.concatenate([sp16, sp16 + np_])
    dstp = _tile_pad_idx(dst, _TILES, _CHUNK, np_)
    acc_cat = _make_agg_cols(np_, e, W1.shape[1] // 2)(hs_cat, srcp, dstp)
    acc0, acc1 = acc_cat[:np_], acc_cat[np_:]

    hs2_cat = _tc2(acc0, acc1, dinv16, b1.reshape(1, -1), W2)
    half_rows = spe.shape[0] // 2
    spe2 = jnp.concatenate([spe[:half_rows], spe[half_rows:] + np_])
    p_cat = _make_agg_edges(np_, e, W2.shape[1])(hs2_cat, spe2, dpe)
    p0, p1 = p_cat[:np_], p_cat[np_:]
    return _tc3(p0, p1, hs2_cat[:np_], dinv16, b2.reshape(1, -1))[:n]
